# baseline (speedup 1.0000x reference)
import functools

import jax
import jax.numpy as jnp
from jax import lax
from jax.experimental import pallas as pl
from jax.experimental.pallas import tpu as pltpu
from jax.experimental.pallas import tpu_sc as plsc

N = 10000
E = 320000
D_IN = 128
H1 = 128
H2 = 16
N_CLASSES = 40

NC, NS, LANES = 2, 16, 16
NW = NC * NS
C = 128
N_CHUNKS = 80
EPAD = NW * C * N_CHUNKS
HF = H1 // 2
N_CHUNKS_FS = EPAD // (NS * C)
NPAD = 10240
RPS = NPAD // NS
BR = 2048
DEGW = 16


def _sc_mesh():
    return plsc.VectorSubcoreMesh(core_axis_name="c", subcore_axis_name="s",
                                  num_cores=NC, num_subcores=NS)


def _gs_pipeline(u_hbm, src_v, dst_v, acc_sh, rows, sem_g, sem_s, n_chunks,
                 k_pipe, drain_hbm=None):
    if drain_hbm is None:
        drain_hbm = u_hbm
    nbuf = 2 * k_pipe
    assert n_chunks % nbuf == 0

    def body(g, carry):
        for t in range(2):
            base = (g * 2 + t) * k_pipe
            tb = t * k_pipe

            @pl.when(g > 0)
            def _(t=t, tb=tb):
                for b in range(k_pipe):
                    pltpu.make_async_copy(drain_hbm.at[pl.ds(0, C)],
                                          rows[tb + b], sem_s[t]).wait()

            gcps = [
                pltpu.async_copy(u_hbm.at[src_v.at[base + b]], rows[tb + b],
                                 sem_g[t])
                for b in range(k_pipe)
            ]
            for b in range(k_pipe):
                gcps[b].wait()
            for b in range(k_pipe):
                pltpu.async_copy(rows[tb + b], acc_sh.at[dst_v.at[base + b]],
                                 sem_s[t], add=True)
        return carry

    lax.fori_loop(0, n_chunks // nbuf, body, 0)
    for t in range(2):
        for b in range(k_pipe):
            pltpu.make_async_copy(drain_hbm.at[pl.ds(0, C)],
                                  rows[t * k_pipe + b], sem_s[t]).wait()


def _edge_agg(h, k_pipe):
    nbuf = 2 * k_pipe

    @functools.partial(
        pl.kernel,
        out_type=jax.ShapeDtypeStruct((NC, NPAD, h), jnp.float32),
        mesh=_sc_mesh(),
        scratch_types=[
            pltpu.VMEM_SHARED((NPAD, h), jnp.float32),
            pltpu.VMEM_SHARED((NPAD, h), jnp.float32),
            pltpu.VMEM((N_CHUNKS, C), jnp.int32),
            pltpu.VMEM((N_CHUNKS, C), jnp.int32),
        ] + [pltpu.VMEM((C, h), jnp.float32) for _ in range(nbuf)]
          + [pltpu.SemaphoreType.DMA for _ in range(4)],
        compiler_params=pltpu.CompilerParams(use_tc_tiling_on_sc=False),
    )
    def k(u_hbm, src_hbm, dst_hbm, zeros_hbm, out_hbm, table_sh, acc_sh,
          src_v, dst_v, *bufs_sems):
        rows = bufs_sems[:nbuf]
        sem_g = bufs_sems[nbuf:nbuf + 2]
        sem_s = bufs_sems[nbuf + 2:nbuf + 4]
        c = lax.axis_index("c")
        s = lax.axis_index("s")
        pltpu.sync_copy(u_hbm.at[pl.ds(s * RPS, RPS)],
                        table_sh.at[pl.ds(s * RPS, RPS)])
        pltpu.sync_copy(zeros_hbm.at[pl.ds(s * RPS, RPS)],
                        acc_sh.at[pl.ds(s * RPS, RPS)])
        pltpu.sync_copy(src_hbm.at[c, s], src_v)
        pltpu.sync_copy(dst_hbm.at[c, s], dst_v)
        plsc.subcore_barrier()
        _gs_pipeline(table_sh, src_v, dst_v, acc_sh, rows, sem_g, sem_s,
                     N_CHUNKS, k_pipe, drain_hbm=u_hbm)
        plsc.subcore_barrier()
        pltpu.sync_copy(acc_sh.at[pl.ds(s * RPS, RPS)],
                        out_hbm.at[c, pl.ds(s * RPS, RPS)])

    return k


def _edge_agg_fs(k_pipe):
    nbuf = 2 * k_pipe

    quarter = N_CHUNKS_FS // 4

    @functools.partial(
        pl.kernel,
        out_type=jax.ShapeDtypeStruct((NC, NPAD, HF), jnp.float32),
        mesh=_sc_mesh(),
        scratch_types=[
            pltpu.VMEM_SHARED((NPAD, HF), jnp.float32),
            pltpu.VMEM_SHARED((NPAD, HF), jnp.float32),
            pltpu.VMEM((quarter, C), jnp.int32),
            pltpu.VMEM((quarter, C), jnp.int32),
        ] + [pltpu.VMEM((C, HF), jnp.float32) for _ in range(nbuf)]
          + [pltpu.SemaphoreType.DMA for _ in range(4)],
        compiler_params=pltpu.CompilerParams(use_tc_tiling_on_sc=False),
    )
    def k(ucat_hbm, src_hbm, dst_hbm, zeros_hbm, out_hbm, table_sh, acc_sh,
          src_v, dst_v, *bufs_sems):
        rows = bufs_sems[:nbuf]
        sem_g = bufs_sems[nbuf:nbuf + 2]
        sem_s = bufs_sems[nbuf + 2:nbuf + 4]
        c = lax.axis_index("c")
        s = lax.axis_index("s")
        pltpu.sync_copy(ucat_hbm.at[pl.ds(c * NPAD + s * RPS, RPS)],
                        table_sh.at[pl.ds(s * RPS, RPS)])
        pltpu.sync_copy(zeros_hbm.at[pl.ds(s * RPS, RPS)],
                        acc_sh.at[pl.ds(s * RPS, RPS)])
        plsc.subcore_barrier()
        for ih in range(4):
            pltpu.sync_copy(src_hbm.at[s, pl.ds(ih * quarter, quarter)],
                            src_v)
            pltpu.sync_copy(dst_hbm.at[s, pl.ds(ih * quarter, quarter)],
                            dst_v)
            _gs_pipeline(table_sh, src_v, dst_v, acc_sh, rows, sem_g, sem_s,
                         quarter, k_pipe, drain_hbm=ucat_hbm)
        plsc.subcore_barrier()
        pltpu.sync_copy(acc_sh.at[pl.ds(s * RPS, RPS)],
                        out_hbm.at[c, pl.ds(s * RPS, RPS)])

    return k


def _edge_agg_c2(k_pipe):
    nbuf = 2 * k_pipe
    CHR = 160

    @functools.partial(
        pl.kernel,
        out_type=(jax.ShapeDtypeStruct((NC, NPAD, H2), jnp.float32),
                  jax.ShapeDtypeStruct((NPAD, H2), jnp.float32)),
        mesh=_sc_mesh(),
        scratch_types=[
            pltpu.VMEM_SHARED((NPAD, H2), jnp.float32),
            pltpu.VMEM_SHARED((NPAD, H2), jnp.float32),
            pltpu.VMEM((N_CHUNKS, C), jnp.int32),
            pltpu.VMEM((N_CHUNKS, C), jnp.int32),
            pltpu.VMEM((CHR, H2), jnp.float32),
            pltpu.VMEM((CHR, H2), jnp.float32),
            pltpu.VMEM((CHR, H2), jnp.float32),
            pltpu.VMEM((CHR, H2), jnp.float32),
            pltpu.VMEM((CHR, H2), jnp.float32),
            pltpu.VMEM((1, H2), jnp.float32),
        ] + [pltpu.VMEM((C, H2), jnp.float32) for _ in range(nbuf)]
          + [pltpu.SemaphoreType.DMA for _ in range(4)],
        compiler_params=pltpu.CompilerParams(use_tc_tiling_on_sc=False),
    )
    def k(s2_hbm, u2_hbm, dinv16_hbm, b2_hbm, src_hbm, dst_hbm, zeros_hbm,
          s3_hbm, u3_hbm, table_sh, acc_sh, src_v, dst_v, sa, sb, u2v, dv,
          u3v, b2v, *bufs_sems):
        rows = bufs_sems[:nbuf]
        sem_g = bufs_sems[nbuf:nbuf + 2]
        sem_s = bufs_sems[nbuf + 2:nbuf + 4]
        c = lax.axis_index("c")
        s = lax.axis_index("s")
        pltpu.sync_copy(zeros_hbm.at[pl.ds(s * RPS, RPS)],
                        acc_sh.at[pl.ds(s * RPS, RPS)])
        pltpu.sync_copy(src_hbm.at[c, s], src_v)
        pltpu.sync_copy(dst_hbm.at[c, s], dst_v)
        pltpu.sync_copy(b2_hbm, b2v)
        b2vec = b2v[0, :]
        for q in range(RPS // CHR):
            base = s * RPS + q * CHR
            pltpu.sync_copy(s2_hbm.at[0, pl.ds(base, CHR)], sa)
            pltpu.sync_copy(s2_hbm.at[1, pl.ds(base, CHR)], sb)
            pltpu.sync_copy(u2_hbm.at[pl.ds(base, CHR)], u2v)
            pltpu.sync_copy(dinv16_hbm.at[pl.ds(base, CHR)], dv)

            def cbody(i, carry):
                for uu in range(8):
                    r = i * 8 + uu
                    d = dv[r, :]
                    pre = sa[r, :] + sb[r, :] + u2v[r, :]
                    t = jnp.maximum(d * pre + b2vec, 0.0)
                    u3v[r, :] = d * t
                return carry

            lax.fori_loop(0, CHR // 8, cbody, 0)
            pltpu.sync_copy(u3v, table_sh.at[pl.ds(base, CHR)])

            @pl.when(c == 0)
            def _(base=base):
                pltpu.sync_copy(u3v, u3_hbm.at[pl.ds(base, CHR)])

        plsc.subcore_barrier()
        _gs_pipeline(table_sh, src_v, dst_v, acc_sh, rows, sem_g, sem_s,
                     N_CHUNKS, k_pipe, drain_hbm=u2_hbm)
        plsc.subcore_barrier()
        pltpu.sync_copy(acc_sh.at[pl.ds(s * RPS, RPS)],
                        s3_hbm.at[c, pl.ds(s * RPS, RPS)])

    return k


def _degree():
    @functools.partial(
        pl.kernel,
        out_type=jax.ShapeDtypeStruct((NC, NPAD, DEGW), jnp.float32),
        mesh=_sc_mesh(),
        scratch_types=[
            pltpu.VMEM_SHARED((NPAD, DEGW), jnp.float32),
            pltpu.VMEM((N_CHUNKS, C), jnp.int32),
            pltpu.VMEM((C, DEGW), jnp.float32),
            pltpu.SemaphoreType.DMA,
        ],
        compiler_params=pltpu.CompilerParams(use_tc_tiling_on_sc=False),
    )
    def k(ones_hbm, dst_hbm, zeros_hbm, out_hbm, acc_sh, dst_v, ones_v, sem):
        c = lax.axis_index("c")
        s = lax.axis_index("s")
        pltpu.sync_copy(zeros_hbm.at[pl.ds(s * RPS, RPS)],
                        acc_sh.at[pl.ds(s * RPS, RPS)])
        pltpu.sync_copy(dst_hbm.at[c, s], dst_v)
        pltpu.sync_copy(ones_hbm, ones_v)
        plsc.subcore_barrier()

        def body(j, carry):
            pltpu.sync_copy(ones_v, acc_sh.at[dst_v.at[j]], add=True)
            return carry

        lax.fori_loop(0, N_CHUNKS, body, 0)
        plsc.subcore_barrier()
        pltpu.sync_copy(acc_sh.at[pl.ds(s * RPS, RPS)],
                        out_hbm.at[c, pl.ds(s * RPS, RPS)])

    return k


def _rb(bs):
    nd = len(bs)
    ri = nd - 2 if nd >= 2 else 0

    def imap(i, _nd=nd, _ri=ri):
        idx = [0] * _nd
        idx[_ri] = i
        return tuple(idx)

    return pl.BlockSpec(bs, imap)


def _fb(bs):
    return pl.BlockSpec(bs, lambda i, _nd=len(bs): (0,) * _nd)


def _row_grid(*block_shapes):
    return [None if bs is None else _rb(bs) for bs in block_shapes]


def _mm_scale_body(x_ref, w1_ref, degp_ref, u1_ref, dinv_ref, dinv16_ref):
    deg = 1.0 + degp_ref[0, :, 0:1] + degp_ref[1, :, 0:1]
    dinv = lax.rsqrt(deg)
    dinv_ref[...] = dinv
    dinv16_ref[...] = jnp.broadcast_to(dinv, (dinv.shape[0], H2))
    u = dinv * jnp.dot(x_ref[...], w1_ref[...],
                       preferred_element_type=jnp.float32)
    u1_ref[0] = u[:, :HF]
    u1_ref[1] = u[:, HF:]


def _combine1_body(s1_ref, u1_ref, dinv_ref, b1_ref, w2_ref, u2_ref):
    dinv = dinv_ref[...]
    pre = s1_ref[...] + u1_ref[...]
    agg = jnp.concatenate([pre[0], pre[1]], axis=1)
    h = dinv * agg + b1_ref[...]
    h = jnp.maximum(h, 0.0)
    u2_ref[...] = dinv * jnp.dot(h, w2_ref[...],
                                 preferred_element_type=jnp.float32)


def _combine2_body(s2_ref, u2_ref, dinv_ref, b2_ref, u3_ref):
    dinv = dinv_ref[...]
    h = dinv * (s2_ref[0] + s2_ref[1] + u2_ref[...]) + b2_ref[...]
    u3_ref[...] = dinv * jnp.maximum(h, 0.0)


def _final_body(s3_ref, u3_ref, dinv_ref, b3_ref, w3_ref, out_ref):
    agg = dinv_ref[...] * (s3_ref[0] + s3_ref[1] + u3_ref[...])
    z = jnp.dot(agg, w3_ref[...], preferred_element_type=jnp.float32)
    z = z + b3_ref[...]
    zmax = jnp.max(z, axis=1, keepdims=True)
    zs = z - zmax
    out_ref[...] = zs - jnp.log(jnp.sum(jnp.exp(zs), axis=1, keepdims=True))


def _tc_call(body, in_specs, out_specs, out_shape):
    return pl.pallas_call(
        body,
        grid=(NPAD // BR,),
        in_specs=in_specs,
        out_specs=out_specs,
        out_shape=out_shape,
    )


def kernel(x, edge_index, W1, b1, W2, b2, W3, b3):
    f32 = jnp.float32
    src = edge_index[0]
    dst = edge_index[1]
    pad = jnp.full((EPAD - E,), N, jnp.int32)
    src_flat = jnp.concatenate([src, pad])
    dst_flat = jnp.concatenate([dst, pad])
    srcp = src_flat.reshape(NC, NS, N_CHUNKS, C)
    dstp = dst_flat.reshape(NC, NS, N_CHUNKS, C)
    src_fs = src_flat.reshape(NS, N_CHUNKS_FS, C)
    dst_fs = dst_flat.reshape(NS, N_CHUNKS_FS, C)

    xp = jnp.zeros((NPAD, D_IN), f32).at[:N].set(x)
    z64 = jnp.zeros((NPAD, HF), f32)
    z16 = jnp.zeros((NPAD, H2), f32)
    zdeg = jnp.zeros((NPAD, DEGW), f32)
    ones = jnp.ones((C, DEGW), f32)

    degp = _degree()(ones, dstp, zdeg)

    u1s, dinv, dinv16 = _tc_call(
        _mm_scale_body,
        [_rb((BR, D_IN)), _fb((D_IN, H1)), _rb((NC, BR, DEGW))],
        (_rb((NC, BR, HF)), _rb((BR, 1)), _rb((BR, H2))),
        (jax.ShapeDtypeStruct((NC, NPAD, HF), f32),
         jax.ShapeDtypeStruct((NPAD, 1), f32),
         jax.ShapeDtypeStruct((NPAD, H2), f32)),
    )(xp, W1, degp)

    s1 = _edge_agg_fs(2)(u1s.reshape(NC * NPAD, HF), src_fs, dst_fs, z64)
    u2 = _tc_call(
        _combine1_body,
        [_rb((NC, BR, HF)), _rb((NC, BR, HF)), _rb((BR, 1)), _fb((1, H1)),
         _fb((H1, H2))],
        _rb((BR, H2)),
        jax.ShapeDtypeStruct((NPAD, H2), f32),
    )(s1, u1s, dinv, b1.reshape(1, H1), W2)

    s2 = _edge_agg(H2, 8)(u2, srcp, dstp, z16)
    s3, u3 = _edge_agg_c2(8)(s2, u2, dinv16, b2.reshape(1, H2), srcp, dstp,
                             z16)
    out = _tc_call(
        _final_body,
        [_rb((NC, BR, H2)), _rb((BR, H2)), _rb((BR, 1)),
         _fb((1, N_CLASSES)), _fb((H2, N_CLASSES))],
        _rb((BR, N_CLASSES)),
        jax.ShapeDtypeStruct((NPAD, N_CLASSES), f32),
    )(s3, u3, dinv, b3.reshape(1, N_CLASSES), W3)

    return out[:N]


# speedup vs baseline: 1.0266x; 1.0266x over previous
import functools

import jax
import jax.numpy as jnp
from jax import lax
from jax.experimental import pallas as pl
from jax.experimental.pallas import tpu as pltpu
from jax.experimental.pallas import tpu_sc as plsc

N = 10000
E = 320000
D_IN = 128
H1 = 128
H2 = 16
N_CLASSES = 40

NC, NS, LANES = 2, 16, 16
NW = NC * NS
C = 128
N_CHUNKS = 80
EPAD = NW * C * N_CHUNKS
HF = H1 // 2
N_CHUNKS_FS = EPAD // (NS * C)
NPAD = 10240
RPS = NPAD // NS
BR = 2048
DEGW = 8


def _sc_mesh():
    return plsc.VectorSubcoreMesh(core_axis_name="c", subcore_axis_name="s",
                                  num_cores=NC, num_subcores=NS)


def _gs_pipeline(u_hbm, src_v, dst_v, acc_sh, rows, sem_g, sem_s, n_chunks,
                 k_pipe, drain_hbm=None):
    if drain_hbm is None:
        drain_hbm = u_hbm
    nbuf = 2 * k_pipe
    assert n_chunks % nbuf == 0

    def body(g, carry):
        for t in range(2):
            base = (g * 2 + t) * k_pipe
            tb = t * k_pipe

            @pl.when(g > 0)
            def _(t=t, tb=tb):
                for b in range(k_pipe):
                    pltpu.make_async_copy(drain_hbm.at[pl.ds(0, C)],
                                          rows[tb + b], sem_s[t]).wait()

            gcps = [
                pltpu.async_copy(u_hbm.at[src_v.at[base + b]], rows[tb + b],
                                 sem_g[t])
                for b in range(k_pipe)
            ]
            for b in range(k_pipe):
                gcps[b].wait()
            for b in range(k_pipe):
                pltpu.async_copy(rows[tb + b], acc_sh.at[dst_v.at[base + b]],
                                 sem_s[t], add=True)
        return carry

    lax.fori_loop(0, n_chunks // nbuf, body, 0)
    for t in range(2):
        for b in range(k_pipe):
            pltpu.make_async_copy(drain_hbm.at[pl.ds(0, C)],
                                  rows[t * k_pipe + b], sem_s[t]).wait()


def _edge_agg(h, k_pipe):
    nbuf = 2 * k_pipe

    @functools.partial(
        pl.kernel,
        out_type=jax.ShapeDtypeStruct((NC, NPAD, h), jnp.float32),
        mesh=_sc_mesh(),
        scratch_types=[
            pltpu.VMEM_SHARED((NPAD, h), jnp.float32),
            pltpu.VMEM_SHARED((NPAD, h), jnp.float32),
            pltpu.VMEM((N_CHUNKS, C), jnp.int32),
            pltpu.VMEM((N_CHUNKS, C), jnp.int32),
        ] + [pltpu.VMEM((C, h), jnp.float32) for _ in range(nbuf)]
          + [pltpu.SemaphoreType.DMA for _ in range(4)],
        compiler_params=pltpu.CompilerParams(use_tc_tiling_on_sc=False),
    )
    def k(u_hbm, src_hbm, dst_hbm, zeros_hbm, out_hbm, table_sh, acc_sh,
          src_v, dst_v, *bufs_sems):
        rows = bufs_sems[:nbuf]
        sem_g = bufs_sems[nbuf:nbuf + 2]
        sem_s = bufs_sems[nbuf + 2:nbuf + 4]
        c = lax.axis_index("c")
        s = lax.axis_index("s")
        pltpu.sync_copy(u_hbm.at[pl.ds(s * RPS, RPS)],
                        table_sh.at[pl.ds(s * RPS, RPS)])
        pltpu.sync_copy(zeros_hbm.at[pl.ds(s * RPS, RPS)],
                        acc_sh.at[pl.ds(s * RPS, RPS)])
        pltpu.sync_copy(src_hbm.at[c, s], src_v)
        pltpu.sync_copy(dst_hbm.at[c, s], dst_v)
        plsc.subcore_barrier()
        _gs_pipeline(table_sh, src_v, dst_v, acc_sh, rows, sem_g, sem_s,
                     N_CHUNKS, k_pipe, drain_hbm=u_hbm)
        plsc.subcore_barrier()
        pltpu.sync_copy(acc_sh.at[pl.ds(s * RPS, RPS)],
                        out_hbm.at[c, pl.ds(s * RPS, RPS)])

    return k


def _edge_agg_fs(k_pipe):
    nbuf = 2 * k_pipe

    quarter = N_CHUNKS_FS // 4

    @functools.partial(
        pl.kernel,
        out_type=jax.ShapeDtypeStruct((NC, NPAD, HF), jnp.float32),
        mesh=_sc_mesh(),
        scratch_types=[
            pltpu.VMEM_SHARED((NPAD, HF), jnp.float32),
            pltpu.VMEM_SHARED((NPAD, HF), jnp.float32),
            pltpu.VMEM((quarter, C), jnp.int32),
            pltpu.VMEM((quarter, C), jnp.int32),
        ] + [pltpu.VMEM((C, HF), jnp.float32) for _ in range(nbuf)]
          + [pltpu.SemaphoreType.DMA for _ in range(4)],
        compiler_params=pltpu.CompilerParams(use_tc_tiling_on_sc=False),
    )
    def k(ucat_hbm, src_hbm, dst_hbm, zeros_hbm, out_hbm, table_sh, acc_sh,
          src_v, dst_v, *bufs_sems):
        rows = bufs_sems[:nbuf]
        sem_g = bufs_sems[nbuf:nbuf + 2]
        sem_s = bufs_sems[nbuf + 2:nbuf + 4]
        c = lax.axis_index("c")
        s = lax.axis_index("s")
        pltpu.sync_copy(ucat_hbm.at[pl.ds(c * NPAD + s * RPS, RPS)],
                        table_sh.at[pl.ds(s * RPS, RPS)])
        pltpu.sync_copy(zeros_hbm.at[pl.ds(s * RPS, RPS)],
                        acc_sh.at[pl.ds(s * RPS, RPS)])
        plsc.subcore_barrier()
        for ih in range(4):
            pltpu.sync_copy(src_hbm.at[s, pl.ds(ih * quarter, quarter)],
                            src_v)
            pltpu.sync_copy(dst_hbm.at[s, pl.ds(ih * quarter, quarter)],
                            dst_v)
            _gs_pipeline(table_sh, src_v, dst_v, acc_sh, rows, sem_g, sem_s,
                         quarter, k_pipe, drain_hbm=ucat_hbm)
        plsc.subcore_barrier()
        pltpu.sync_copy(acc_sh.at[pl.ds(s * RPS, RPS)],
                        out_hbm.at[c, pl.ds(s * RPS, RPS)])

    return k


def _edge_agg_c2(k_pipe):
    nbuf = 2 * k_pipe
    CHR = RPS

    @functools.partial(
        pl.kernel,
        out_type=(jax.ShapeDtypeStruct((NC, NPAD, H2), jnp.float32),
                  jax.ShapeDtypeStruct((NPAD, H2), jnp.float32)),
        mesh=_sc_mesh(),
        scratch_types=[
            pltpu.VMEM_SHARED((NPAD, H2), jnp.float32),
            pltpu.VMEM_SHARED((NPAD, H2), jnp.float32),
            pltpu.VMEM((N_CHUNKS, C), jnp.int32),
            pltpu.VMEM((N_CHUNKS, C), jnp.int32),
            pltpu.VMEM((CHR, H2), jnp.float32),
            pltpu.VMEM((CHR, H2), jnp.float32),
            pltpu.VMEM((CHR, H2), jnp.float32),
            pltpu.VMEM((CHR, H2), jnp.float32),
            pltpu.VMEM((CHR, H2), jnp.float32),
            pltpu.VMEM((1, H2), jnp.float32),
        ] + [pltpu.VMEM((C, H2), jnp.float32) for _ in range(nbuf)]
          + [pltpu.SemaphoreType.DMA for _ in range(4)],
        compiler_params=pltpu.CompilerParams(use_tc_tiling_on_sc=False),
    )
    def k(s2_hbm, u2_hbm, dinv16_hbm, b2_hbm, src_hbm, dst_hbm, zeros_hbm,
          s3_hbm, u3_hbm, table_sh, acc_sh, src_v, dst_v, sa, sb, u2v, dv,
          u3v, b2v, *bufs_sems):
        rows = bufs_sems[:nbuf]
        sem_g = bufs_sems[nbuf:nbuf + 2]
        sem_s = bufs_sems[nbuf + 2:nbuf + 4]
        c = lax.axis_index("c")
        s = lax.axis_index("s")
        pltpu.sync_copy(zeros_hbm.at[pl.ds(s * RPS, RPS)],
                        acc_sh.at[pl.ds(s * RPS, RPS)])
        pltpu.sync_copy(src_hbm.at[c, s], src_v)
        pltpu.sync_copy(dst_hbm.at[c, s], dst_v)
        pltpu.sync_copy(b2_hbm, b2v)
        b2vec = b2v[0, :]
        base = s * RPS
        cps = [pltpu.async_copy(s2_hbm.at[0, pl.ds(base, CHR)], sa, sem_g[0]),
               pltpu.async_copy(s2_hbm.at[1, pl.ds(base, CHR)], sb, sem_g[0]),
               pltpu.async_copy(u2_hbm.at[pl.ds(base, CHR)], u2v, sem_g[0]),
               pltpu.async_copy(dinv16_hbm.at[pl.ds(base, CHR)], dv,
                                sem_g[0])]
        for cp in cps:
            cp.wait()

        def cbody(i, carry):
            for uu in range(8):
                r = i * 8 + uu
                d = dv[r, :]
                pre = sa[r, :] + sb[r, :] + u2v[r, :]
                t = jnp.maximum(d * pre + b2vec, 0.0)
                u3v[r, :] = d * t
            return carry

        lax.fori_loop(0, CHR // 8, cbody, 0)
        pltpu.sync_copy(u3v, table_sh.at[pl.ds(base, CHR)])

        @pl.when(c == 0)
        def _():
            pltpu.sync_copy(u3v, u3_hbm.at[pl.ds(base, CHR)])

        plsc.subcore_barrier()
        _gs_pipeline(table_sh, src_v, dst_v, acc_sh, rows, sem_g, sem_s,
                     N_CHUNKS, k_pipe, drain_hbm=u2_hbm)
        plsc.subcore_barrier()
        pltpu.sync_copy(acc_sh.at[pl.ds(s * RPS, RPS)],
                        s3_hbm.at[c, pl.ds(s * RPS, RPS)])

    return k


def _degree():
    @functools.partial(
        pl.kernel,
        out_type=jax.ShapeDtypeStruct((NC, NPAD, DEGW), jnp.float32),
        mesh=_sc_mesh(),
        scratch_types=[
            pltpu.VMEM_SHARED((NPAD, DEGW), jnp.float32),
            pltpu.VMEM((N_CHUNKS, C), jnp.int32),
            pltpu.VMEM((C, DEGW), jnp.float32),
            pltpu.SemaphoreType.DMA,
        ],
        compiler_params=pltpu.CompilerParams(use_tc_tiling_on_sc=False),
    )
    def k(ones_hbm, dst_hbm, zeros_hbm, out_hbm, acc_sh, dst_v, ones_v, sem):
        c = lax.axis_index("c")
        s = lax.axis_index("s")
        pltpu.sync_copy(zeros_hbm.at[pl.ds(s * RPS, RPS)],
                        acc_sh.at[pl.ds(s * RPS, RPS)])
        pltpu.sync_copy(dst_hbm.at[c, s], dst_v)
        pltpu.sync_copy(ones_hbm, ones_v)
        plsc.subcore_barrier()

        def body(j, carry):
            pltpu.sync_copy(ones_v, acc_sh.at[dst_v.at[j]], add=True)
            return carry

        lax.fori_loop(0, N_CHUNKS, body, 0)
        plsc.subcore_barrier()
        pltpu.sync_copy(acc_sh.at[pl.ds(s * RPS, RPS)],
                        out_hbm.at[c, pl.ds(s * RPS, RPS)])

    return k


def _rb(bs):
    nd = len(bs)
    ri = nd - 2 if nd >= 2 else 0

    def imap(i, _nd=nd, _ri=ri):
        idx = [0] * _nd
        idx[_ri] = i
        return tuple(idx)

    return pl.BlockSpec(bs, imap)


def _fb(bs):
    return pl.BlockSpec(bs, lambda i, _nd=len(bs): (0,) * _nd)


def _row_grid(*block_shapes):
    return [None if bs is None else _rb(bs) for bs in block_shapes]


def _mm_scale_body(x_ref, w1_ref, degp_ref, u1_ref, dinv_ref, dinv16_ref):
    deg = 1.0 + degp_ref[0, :, 0:1] + degp_ref[1, :, 0:1]
    dinv = lax.rsqrt(deg)
    dinv_ref[...] = dinv
    dinv16_ref[...] = jnp.broadcast_to(dinv, (dinv.shape[0], H2))
    u = dinv * jnp.dot(x_ref[...], w1_ref[...],
                       preferred_element_type=jnp.float32)
    u1_ref[0] = u[:, :HF]
    u1_ref[1] = u[:, HF:]


def _combine1_body(s1_ref, u1_ref, dinv_ref, b1_ref, w2_ref, u2_ref):
    dinv = dinv_ref[...]
    pre = s1_ref[...] + u1_ref[...]
    agg = jnp.concatenate([pre[0], pre[1]], axis=1)
    h = dinv * agg + b1_ref[...]
    h = jnp.maximum(h, 0.0)
    u2_ref[...] = dinv * jnp.dot(h, w2_ref[...],
                                 preferred_element_type=jnp.float32)


def _combine2_body(s2_ref, u2_ref, dinv_ref, b2_ref, u3_ref):
    dinv = dinv_ref[...]
    h = dinv * (s2_ref[0] + s2_ref[1] + u2_ref[...]) + b2_ref[...]
    u3_ref[...] = dinv * jnp.maximum(h, 0.0)


def _final_body(s3_ref, u3_ref, dinv_ref, b3_ref, w3_ref, out_ref):
    agg = dinv_ref[...] * (s3_ref[0] + s3_ref[1] + u3_ref[...])
    z = jnp.dot(agg, w3_ref[...], preferred_element_type=jnp.float32)
    z = z + b3_ref[...]
    zmax = jnp.max(z, axis=1, keepdims=True)
    zs = z - zmax
    out_ref[...] = zs - jnp.log(jnp.sum(jnp.exp(zs), axis=1, keepdims=True))


def _tc_call(body, in_specs, out_specs, out_shape):
    return pl.pallas_call(
        body,
        grid=(NPAD // BR,),
        in_specs=in_specs,
        out_specs=out_specs,
        out_shape=out_shape,
    )


def kernel(x, edge_index, W1, b1, W2, b2, W3, b3):
    f32 = jnp.float32
    src = edge_index[0]
    dst = edge_index[1]
    pad = jnp.full((EPAD - E,), N, jnp.int32)
    src_flat = jnp.concatenate([src, pad])
    dst_flat = jnp.concatenate([dst, pad])
    srcp = src_flat.reshape(NC, NS, N_CHUNKS, C)
    dstp = dst_flat.reshape(NC, NS, N_CHUNKS, C)
    src_fs = src_flat.reshape(NS, N_CHUNKS_FS, C)
    dst_fs = dst_flat.reshape(NS, N_CHUNKS_FS, C)

    xp = jnp.zeros((NPAD, D_IN), f32).at[:N].set(x)
    z64 = jnp.zeros((NPAD, HF), f32)
    z16 = jnp.zeros((NPAD, H2), f32)
    zdeg = jnp.zeros((NPAD, DEGW), f32)
    ones = jnp.ones((C, DEGW), f32)

    degp = _degree()(ones, dstp, zdeg)

    u1s, dinv, dinv16 = _tc_call(
        _mm_scale_body,
        [_rb((BR, D_IN)), _fb((D_IN, H1)), _rb((NC, BR, DEGW))],
        (_rb((NC, BR, HF)), _rb((BR, 1)), _rb((BR, H2))),
        (jax.ShapeDtypeStruct((NC, NPAD, HF), f32),
         jax.ShapeDtypeStruct((NPAD, 1), f32),
         jax.ShapeDtypeStruct((NPAD, H2), f32)),
    )(xp, W1, degp)

    s1 = _edge_agg_fs(2)(u1s.reshape(NC * NPAD, HF), src_fs, dst_fs, z64)
    u2 = _tc_call(
        _combine1_body,
        [_rb((NC, BR, HF)), _rb((NC, BR, HF)), _rb((BR, 1)), _fb((1, H1)),
         _fb((H1, H2))],
        _rb((BR, H2)),
        jax.ShapeDtypeStruct((NPAD, H2), f32),
    )(s1, u1s, dinv, b1.reshape(1, H1), W2)

    s2 = _edge_agg(H2, 8)(u2, srcp, dstp, z16)
    s3, u3 = _edge_agg_c2(8)(s2, u2, dinv16, b2.reshape(1, H2), srcp, dstp,
                             z16)
    out = _tc_call(
        _final_body,
        [_rb((NC, BR, H2)), _rb((BR, H2)), _rb((BR, 1)),
         _fb((1, N_CLASSES)), _fb((H2, N_CLASSES))],
        _rb((BR, N_CLASSES)),
        jax.ShapeDtypeStruct((NPAD, N_CLASSES), f32),
    )(s3, u3, dinv, b3.reshape(1, N_CLASSES), W3)

    return out[:N]
